# Initial kernel scaffold; baseline (speedup 1.0000x reference)
import jax
import jax.numpy as jnp
from jax.experimental import pallas as pl


def kernel(spans, span_indices, target_indices, opinion_indices, dep_dis_matrix, W_rel, W_dep):
    raise NotImplementedError("write your pallas kernel here")


# speedup vs baseline: 4.8573x; 4.8573x over previous
import functools
import jax
import jax.numpy as jnp
from jax.experimental import pallas as pl
from jax.experimental.pallas import tpu as pltpu

_BUCKET_BINS = (0, 1, 2, 3, 4, 5, 6, 7, 8, 9, 10, 15, 20, 25, 30, 50, 80)


def _tc_body(si_ref, ti_ref, oi_ref, spans_ref, dep_ref, wr_ref, wd_ref,
             pool_ref, cand_ref, *, nt, no, L, D):
  b = pl.program_id(0)
  P = nt * no
  imax = jnp.iinfo(jnp.int32).max

  t_idx = [ti_ref[b, t] for t in range(nt)]
  o_idx = [oi_ref[b, o] for o in range(no)]
  a_s = [si_ref[t_idx[t], 0] for t in range(nt)]
  b_s = [si_ref[t_idx[t], 1] for t in range(nt)]
  c_s = [si_ref[o_idx[o], 0] for o in range(no)]
  d_s = [si_ref[o_idx[o], 1] for o in range(no)]

  t_of_p = jax.lax.broadcasted_iota(jnp.int32, (P, 1), 0) // no
  o_of_p = jax.lax.broadcasted_iota(jnp.int32, (P, 1), 0) % no
  a_col = jnp.full((P, 1), 0, jnp.int32)
  b_col = jnp.full((P, 1), 0, jnp.int32)
  c_col = jnp.full((P, 1), 0, jnp.int32)
  d_col = jnp.full((P, 1), 0, jnp.int32)
  for t in range(nt):
    a_col = jnp.where(t_of_p == t, a_s[t], a_col)
    b_col = jnp.where(t_of_p == t, b_s[t], b_col)
  for o in range(no):
    c_col = jnp.where(o_of_p == o, c_s[o], c_col)
    d_col = jnp.where(o_of_p == o, d_s[o], d_col)

  rel_dis = jnp.minimum(jnp.abs(b_col - c_col), jnp.abs(a_col - d_col))
  rel_id = jnp.full((P, 1), -1, jnp.int32)
  for bin_v in _BUCKET_BINS:
    rel_id = rel_id + (rel_dis >= bin_v).astype(jnp.int32)

  dep = dep_ref[0]
  row_iota = jax.lax.broadcasted_iota(jnp.int32, (L, 1), 0)
  colmins = []
  for t in range(nt):
    rmask = (row_iota >= a_s[t]) & (row_iota <= b_s[t])
    masked = jnp.where(rmask, dep, imax)
    colmins.append(jnp.min(masked, axis=0, keepdims=True))
  colmin = jnp.concatenate(colmins, axis=0)

  colminP = jnp.concatenate(
      [jnp.broadcast_to(colmin[t:t + 1, :], (no, L)) for t in range(nt)], axis=0)
  col_iota = jax.lax.broadcasted_iota(jnp.int32, (1, L), 1)
  cmaskP = (col_iota >= c_col) & (col_iota <= d_col)
  dep_id = jnp.min(jnp.where(cmaskP, colminP, imax), axis=1, keepdims=True)

  oh_rel = (rel_id == jax.lax.broadcasted_iota(jnp.int32, (P, wr_ref.shape[0]), 1)).astype(jnp.float32)
  oh_dep = (dep_id == jax.lax.broadcasted_iota(jnp.int32, (P, wd_ref.shape[0]), 1)).astype(jnp.float32)
  rel_emb = jnp.dot(oh_rel, wr_ref[...], preferred_element_type=jnp.float32)
  dep_emb = jnp.dot(oh_dep, wd_ref[...], preferred_element_type=jnp.float32)
  pool_ref[0, :, 2 * D:2 * D + 128] = rel_emb
  pool_ref[0, :, 2 * D + 128:2 * D + 256] = dep_emb

  for t in range(nt):
    row = spans_ref[0, pl.ds(t_idx[t], 1), :]
    pool_ref[0, pl.ds(t * no, no), 0:D] = jnp.broadcast_to(row, (no, D))
  o_block = jnp.concatenate(
      [spans_ref[0, pl.ds(o_idx[o], 1), :] for o in range(no)], axis=0)
  for t in range(nt):
    pool_ref[0, pl.ds(t * no, no), D:2 * D] = o_block

  cand_ref[0] = jnp.concatenate([a_col, b_col, c_col, d_col], axis=1)


def kernel(spans, span_indices, target_indices, opinion_indices, dep_dis_matrix, W_rel, W_dep):
  B, S, D = spans.shape
  L = dep_dis_matrix.shape[-1]
  nt = target_indices.shape[1]
  no = opinion_indices.shape[1]
  P = nt * no
  out_dim = 2 * D + W_rel.shape[1] + W_dep.shape[1]

  body = functools.partial(_tc_body, nt=nt, no=no, L=L, D=D)
  pool, cand = pl.pallas_call(
      body,
      grid=(B,),
      in_specs=[
          pl.BlockSpec(memory_space=pltpu.SMEM),
          pl.BlockSpec(memory_space=pltpu.SMEM),
          pl.BlockSpec(memory_space=pltpu.SMEM),
          pl.BlockSpec((1, S, D), lambda b: (b, 0, 0)),
          pl.BlockSpec((1, L, L), lambda b: (b, 0, 0)),
          pl.BlockSpec((W_rel.shape[0], W_rel.shape[1]), lambda b: (0, 0)),
          pl.BlockSpec((W_dep.shape[0], W_dep.shape[1]), lambda b: (0, 0)),
      ],
      out_specs=[
          pl.BlockSpec((1, P, out_dim), lambda b: (b, 0, 0)),
          pl.BlockSpec((1, P, 4), lambda b: (b, 0, 0)),
      ],
      out_shape=[
          jax.ShapeDtypeStruct((B, P, out_dim), jnp.float32),
          jax.ShapeDtypeStruct((B, P, 4), jnp.int32),
      ],
  )(span_indices, target_indices, opinion_indices, spans, dep_dis_matrix, W_rel, W_dep)
  return pool, cand
